# baseline (speedup 1.0000x reference)
import functools

import jax
import jax.numpy as jnp
from jax import lax
from jax.experimental import pallas as pl
from jax.experimental.pallas import tpu as pltpu
from jax.experimental.pallas import tpu_sc as plsc

_INFO = plsc.get_sparse_core_info()
_NC = _INFO.num_cores
_NS = _INFO.num_subcores
_NW = _NC * _NS

_CHUNK = 512
_BLK = _CHUNK // 128
_NB = 4


@functools.partial(jax.jit, static_argnums=(2, 3, 4))
def _sc_gather(xt, table, bsz, h, d):
    n = bsz * h
    nblk = bsz // 128
    sb_per_h = bsz // _CHUNK
    nchunks = n // _CHUNK
    per_w = nchunks // _NW
    dg = d // 8
    assert bsz % _CHUNK == 0 and nchunks % _NW == 0 and d % 8 == 0
    assert per_w % _NB == 0 and per_w >= 2 * _NB
    mesh = plsc.VectorSubcoreMesh(core_axis_name="c", subcore_axis_name="s")

    @functools.partial(
        pl.kernel,
        mesh=mesh,
        out_type=jax.ShapeDtypeStruct((h, dg, nblk, 1024), jnp.float32),
        scratch_types=(
            [pltpu.VMEM((_CHUNK,), jnp.int32) for _ in range(_NB)]
            + [pltpu.VMEM((_CHUNK, d), jnp.float32) for _ in range(_NB)]
            + [pltpu.VMEM((_BLK * d * 128,), jnp.float32) for _ in range(2)]
            + [pltpu.SemaphoreType.DMA for _ in range(2 * _NB + 2)]
        ),
        compiler_params=pltpu.CompilerParams(use_tc_tiling_on_sc=False,
                                             needs_layout_passes=False),
    )
    def k(xt_hbm, tab_hbm, out_hbm, *scratch):
        idx_v = scratch[0:_NB]
        rows_v = scratch[_NB:2 * _NB]
        t_v = scratch[2 * _NB:2 * _NB + 2]
        si = scratch[2 * _NB + 2:3 * _NB + 2]
        sg = scratch[3 * _NB + 2:4 * _NB + 2]
        so = scratch[4 * _NB + 2:4 * _NB + 4]

        wid = lax.axis_index("s") * _NC + lax.axis_index("c")
        c0 = wid * per_w
        lane = lax.iota(jnp.int32, 16)

        def chunk_off(i):
            c = c0 + i
            hh = c // sb_per_h
            sb = c - hh * sb_per_h
            return hh, sb

        def idx_copy(i, p):
            hh, sb = chunk_off(i)
            return pltpu.make_async_copy(
                xt_hbm.at[pl.ds(hh * bsz + sb * _CHUNK, _CHUNK)],
                idx_v[p], si[p])

        def gather_copy(p):
            return pltpu.make_async_copy(tab_hbm.at[idx_v[p]], rows_v[p],
                                         sg[p])

        def out_copy(i, q, g, bb):
            hh, sb = chunk_off(i)
            return pltpu.make_async_copy(
                t_v[q].at[pl.ds(bb * d * 128 + g * 1024, 1024)],
                out_hbm.at[hh, g, sb * _BLK + bb], so[q])

        def transpose(p, q):
            rows = rows_v[p]
            dst = t_v[q]
            for bb in range(_BLK):
                base = [(lane + cg * 16) * 128 + bb * (d * 128)
                        for cg in range(d // 16)]

                def jbody(j, carry):
                    row = bb * 128 + j
                    for cg in range(d // 16):
                        v = rows[row, pl.ds(cg * 16, 16)]
                        plsc.store_scatter(dst, [base[cg] + j], v)
                    return carry

                lax.fori_loop(0, 128, jbody, 0, unroll=8)

        def body(i, p, q, prefetch, start_g, wait_out):
            gather_copy(p).wait()
            if prefetch:
                idx_copy(i + _NB, p).start()
            if start_g:
                p3 = (p + 3) % _NB
                idx_copy(0, p3).wait()
                gather_copy(p3).start()
            if wait_out:
                for g in range(dg):
                    for bb in range(_BLK):
                        out_copy(0, q, g, bb).wait()
            transpose(p, q)
            for g in range(dg):
                for bb in range(_BLK):
                    out_copy(i, q, g, bb).start()

        for b in range(_NB):
            idx_copy(b, b).start()
        for b in range(_NB - 1):
            idx_copy(0, b).wait()
            gather_copy(b).start()

        for b in range(_NB):
            body(b, b, b % 2, prefetch=True, start_g=True,
                 wait_out=(b >= 2))

        def loop(j, carry):
            i = _NB * j
            for b in range(_NB):
                body(i + b, b, b % 2, prefetch=True, start_g=True,
                     wait_out=True)
            return carry

        lax.fori_loop(1, per_w // _NB - 1, loop, 0)

        i0 = per_w - _NB
        for b in range(_NB):
            body(i0 + b, b, b % 2, prefetch=False, start_g=(b < 1),
                 wait_out=True)

        for q in range(2):
            for g in range(dg):
                for bb in range(_BLK):
                    out_copy(0, q, g, bb).wait()

    return k(xt, table)


def kernel(x, table):
    b, h = x.shape
    v, d = table.shape
    xt = x.T.reshape(b * h)
    out5 = _sc_gather(xt, table, b, h, d)
    out5 = out5.reshape(h, d // 8, b // 128, 8, 128)
    return out5.transpose(2, 4, 0, 1, 3).reshape(b, h, d)
